# baseline (speedup 1.0000x reference)
import functools

import jax
import jax.numpy as jnp
from jax import lax
from jax.experimental import pallas as pl
from jax.experimental.pallas import tpu as pltpu, tpu_sc as plsc

_INFO = plsc.get_sparse_core_info()
_NC = _INFO.num_cores
_NS = _INFO.num_subcores
_NW = _NC * _NS
_L = _INFO.num_lanes

_B = 16384
_D = 64
_PER_W = _B // _NW
_CH = 128
_NCHUNK = _PER_W // _CH
_GRP = _CH // _L


@functools.partial(
    pl.kernel,
    mesh=plsc.VectorSubcoreMesh(core_axis_name="c", subcore_axis_name="s"),
    compiler_params=pltpu.CompilerParams(
        needs_layout_passes=False, use_tc_tiling_on_sc=False),
    out_type=jax.ShapeDtypeStruct((_NW, _L), jnp.float32),
    scratch_types=[
        pltpu.VMEM((4, _PER_W), jnp.int32),
        pltpu.VMEM((8,), jnp.int32),
        pltpu.VMEM((8, 2 * _D), jnp.float32),
        pltpu.VMEM((_CH, _D), jnp.float32),
        pltpu.VMEM((_CH, _D), jnp.float32),
        pltpu.VMEM((_CH, _D), jnp.float32),
        pltpu.VMEM((_CH, _D), jnp.float32),
        pltpu.VMEM((_L,), jnp.float32),
        pltpu.SemaphoreType.DMA,
    ],
)
def _gmf_sc(data_hbm, wu_hbm, ww_hbm, wk_hbm, out_hbm,
            idx_all, idx_u, u_rows, w_rows, kw_rows, nw_rows, nkw_rows,
            sum_v, sem):
    wid = lax.axis_index("s") * _NC + lax.axis_index("c")
    base = wid * _PER_W

    pltpu.sync_copy(data_hbm.at[pl.ds(1, 4), pl.ds(base, _PER_W)], idx_all)
    pltpu.sync_copy(data_hbm.at[0, pl.ds(0, 8)], idx_u)
    pltpu.async_copy(wu_hbm.at[idx_u], u_rows, sem).wait()

    iota16 = lax.iota(jnp.int32, _L)

    def lane_bcast(vec, lanev):
        return lax.gather(
            vec, lanev[:, None],
            lax.GatherDimensionNumbers(
                offset_dims=(), collapsed_slice_dims=(0,),
                start_index_map=(0,)),
            (1,),
            mode=lax.GatherScatterMode.PROMISE_IN_BOUNDS)

    def chunk_body(c, total):
        sl = pl.ds(c * _CH, _CH)
        cps = [
            pltpu.async_copy(ww_hbm.at[idx_all.at[1, sl]], w_rows, sem),
            pltpu.async_copy(wk_hbm.at[idx_all.at[0, sl]], kw_rows, sem),
            pltpu.async_copy(ww_hbm.at[idx_all.at[3, sl]], nw_rows, sem),
            pltpu.async_copy(wk_hbm.at[idx_all.at[2, sl]], nkw_rows, sem),
        ]
        for cp in cps:
            cp.wait()

        def dbody(d, carry):
            pos = list(carry[:_GRP])
            neg = list(carry[_GRP:])
            colv = jnp.broadcast_to(d, (_L,))
            dbase = jnp.bitwise_and(d, -_L)
            lanev = jnp.broadcast_to(jnp.bitwise_and(d, _L - 1), (_L,))
            u_lo = lane_bcast(u_rows[0, pl.ds(dbase, _L)], lanev)
            u_hi = lane_bcast(u_rows[0, pl.ds(dbase + _D, _L)], lanev)
            for g in range(_GRP):
                rid = iota16 + g * _L
                pos[g] = (pos[g]
                          + plsc.load_gather(w_rows, [rid, colv]) * u_lo
                          + plsc.load_gather(kw_rows, [rid, colv]) * u_hi)
                neg[g] = (neg[g]
                          + plsc.load_gather(nw_rows, [rid, colv]) * u_lo
                          + plsc.load_gather(nkw_rows, [rid, colv]) * u_hi)
            return tuple(pos) + tuple(neg)

        init = tuple(jnp.zeros((_L,), jnp.float32) for _ in range(2 * _GRP))
        accs = lax.fori_loop(0, _D, dbody, init)
        one = jnp.float32(1.0)
        two = jnp.float32(2.0)

        def sigmoid(x):
            y = one + jnp.exp(-x)
            r = one / y
            r = r * (two - y * r)
            return r * (two - y * r)

        for g in range(_GRP):
            total = total + sigmoid(accs[g]) - sigmoid(accs[_GRP + g])
        return total

    total = lax.fori_loop(0, _NCHUNK, chunk_body,
                          jnp.zeros((_L,), jnp.float32))

    sum_v[...] = total
    pltpu.sync_copy(sum_v, out_hbm.at[wid])


def kernel(data, W_user, W_writer, W_keywd):
    data = data.astype(jnp.int32)
    partials = _gmf_sc(data, W_user, W_writer, W_keywd)
    return jnp.sum(partials)
